# baseline (device time: 1199024 ns/iter reference)
import os

import jax
import jax.numpy as jnp
from jax import lax
from jax.experimental import pallas as pl
from jax.experimental.pallas import tpu as pltpu

N_DEV = 4
M, K_SH, N = 4096, 1024, 8192
MC = M // N_DEV
RH = MC // 2
SUB = int(os.environ.get("KERNEL_SUB", "2"))
SM = RH // SUB
TMG = min(SM, 256)
N_SEM = 2 * (N_DEV - 1) * SUB

_SKIP_GEMM = bool(os.environ.get("KERNEL_SKIP_GEMM"))
_SKIP_ADD = bool(os.environ.get("KERNEL_SKIP_ADD"))
_SKIP_AG = bool(os.environ.get("KERNEL_SKIP_AG"))
_BARRIER_ONLY = bool(os.environ.get("KERNEL_BARRIER_ONLY"))


def kernel(x, w_mat):
    def body(x_ref, w_ref, out_ref, partial_ref,
             xt, gacc, aacc, abuf,
             sendA, recvA, sendB, recvB, cp_sems):
        i = lax.axis_index("i")
        left = jnp.mod(i - 1, N_DEV)
        right = jnp.mod(i + 1, N_DEV)
        diag = jnp.mod(i + 2, N_DEV)

        barrier = pltpu.get_barrier_semaphore()
        for nbr in (left, right):
            pl.semaphore_signal(
                barrier, inc=1,
                device_id=(nbr,), device_id_type=pl.DeviceIdType.MESH,
            )
        pl.semaphore_wait(barrier, 2)

        if _BARRIER_ONLY:
            return

        def gemm_tile(c, t):
            if _SKIP_GEMM:
                return
            row = c * MC + t * TMG
            ld = pltpu.make_async_copy(
                x_ref.at[pl.ds(row, TMG), :], xt, cp_sems.at[0])
            ld.start()
            ld.wait()
            gacc[...] = jnp.dot(
                xt[...], w_ref[...], preferred_element_type=jnp.float32)
            st = pltpu.make_async_copy(
                gacc, partial_ref.at[pl.ds(row, TMG), :], cp_sems.at[0])
            st.start()
            st.wait()

        def gemm_range(c, t0, nt):
            if _SKIP_GEMM:
                return
            lax.fori_loop(t0, t0 + nt, lambda t, _: (gemm_tile(c, t), 0)[1], 0)

        def sub_row(c, roff, u):
            return c * MC + roff + u * SM

        def sub_rdma(src_buf, c, roff, u, slot, ssem, rsem, dev):
            row = sub_row(c, roff, u)
            return pltpu.make_async_remote_copy(
                src_ref=src_buf.at[pl.ds(row, SM), :],
                dst_ref=out_ref.at[pl.ds(row, SM), :],
                send_sem=ssem.at[slot],
                recv_sem=rsem.at[slot],
                device_id=(dev,),
                device_id_type=pl.DeviceIdType.MESH,
            )

        def add_sub(rc, roff, u, do_relu):
            if _SKIP_ADD:
                return
            row = sub_row(rc, roff, u)
            ld_a = pltpu.make_async_copy(
                partial_ref.at[pl.ds(row, SM), :], aacc, cp_sems.at[0])
            ld_b = pltpu.make_async_copy(
                out_ref.at[pl.ds(row, SM), :], abuf, cp_sems.at[1])
            ld_a.start()
            ld_b.start()
            ld_a.wait()
            ld_b.wait()
            val = aacc[...] + abuf[...]
            if do_relu:
                val = jnp.maximum(val, 0.0)
            aacc[...] = val
            st = pltpu.make_async_copy(
                aacc, out_ref.at[pl.ds(row, SM), :], cp_sems.at[0])
            st.start()
            st.wait()

        rd = {}

        for u in range(SUB):
            gemm_tile(left, u)
            rd["A", 0, u] = sub_rdma(partial_ref, left, 0, u, u,
                                     sendA, recvA, right)
            rd["A", 0, u].start()
            gemm_tile(right, SUB + u)
            rd["B", 0, u] = sub_rdma(partial_ref, right, RH, u, u,
                                     sendB, recvB, left)
            rd["B", 0, u].start()

        gemm_range(diag, 0, 2 * SUB)
        pending_gemm = [
            (right, 0, SUB),
            (left, SUB, SUB),
            (i, 0, SUB),
            (i, SUB, SUB),
        ]

        for s in range(N_DEV - 1):
            rcA = jnp.mod(i - 2 - s, N_DEV)
            rcB = jnp.mod(i + 2 + s, N_DEV)
            do_relu = (s == N_DEV - 2)
            for u in range(SUB):
                rd["A", s, u].wait()
                add_sub(rcA, 0, u, do_relu)
                rd["B", s, u].wait()
                add_sub(rcB, RH, u, do_relu)
                if s < N_DEV - 2:
                    slot = (s + 1) * SUB + u
                    rd["A", s + 1, u] = sub_rdma(
                        out_ref, rcA, 0, u, slot, sendA, recvA, right)
                    rd["B", s + 1, u] = sub_rdma(
                        out_ref, rcB, RH, u, slot, sendB, recvB, left)
                else:
                    if _SKIP_AG:
                        continue
                    slot = (N_DEV - 1) * SUB + u
                    rd["GA", 0, u] = sub_rdma(
                        out_ref, i, 0, u, slot, sendA, recvA, right)
                    rd["GB", 0, u] = sub_rdma(
                        out_ref, i, RH, u, slot, sendB, recvB, left)
                    rd["GA", 0, u].start()
                    rd["GB", 0, u].start()
                    continue
                rd["A", s + 1, u].start()
                rd["B", s + 1, u].start()
                if pending_gemm:
                    gemm_range(*pending_gemm.pop(0))

        for s in range(0 if _SKIP_AG else N_DEV - 1):
            for u in range(SUB):
                rd["GA", s, u].wait()
                rd["GB", s, u].wait()
                if s < N_DEV - 2:
                    slot = (N_DEV - 1 + s + 1) * SUB + u
                    rd["GA", s + 1, u] = sub_rdma(
                        out_ref, jnp.mod(i - 1 - s, N_DEV), 0, u, slot,
                        sendA, recvA, right)
                    rd["GB", s + 1, u] = sub_rdma(
                        out_ref, jnp.mod(i + 1 + s, N_DEV), RH, u, slot,
                        sendB, recvB, left)
                    rd["GA", s + 1, u].start()
                    rd["GB", s + 1, u].start()

    out, _partial = pl.pallas_call(
        body,
        out_shape=[
            jax.ShapeDtypeStruct((M, N), jnp.float32),
            jax.ShapeDtypeStruct((M, N), jnp.float32),
        ],
        in_specs=[
            pl.BlockSpec(memory_space=pl.ANY),
            pl.BlockSpec(memory_space=pltpu.VMEM),
        ],
        out_specs=[
            pl.BlockSpec(memory_space=pl.ANY),
            pl.BlockSpec(memory_space=pl.ANY),
        ],
        scratch_shapes=[
            pltpu.VMEM((TMG, K_SH), jnp.float32),
            pltpu.VMEM((TMG, N), jnp.float32),
            pltpu.VMEM((8 if _SKIP_ADD else SM, N), jnp.float32),
            pltpu.VMEM((8 if _SKIP_ADD else SM, N), jnp.float32),
            pltpu.SemaphoreType.DMA((N_SEM,)),
            pltpu.SemaphoreType.DMA((N_SEM,)),
            pltpu.SemaphoreType.DMA((N_SEM,)),
            pltpu.SemaphoreType.DMA((N_SEM,)),
            pltpu.SemaphoreType.DMA((2,)),
        ],
        compiler_params=pltpu.CompilerParams(
            collective_id=0,
            vmem_limit_bytes=62 * 1024 * 1024,
        ),
    )(x, w_mat)
    return out
